# Initial kernel scaffold; baseline (speedup 1.0000x reference)
import jax
import jax.numpy as jnp
from jax.experimental import pallas as pl


def kernel(idx_cat0, idx_cat1, idx_cat2, idx_cat3, emb_cat0, emb_cat1, emb_cat2, emb_cat3):
    raise NotImplementedError("write your pallas kernel here")


# speedup vs baseline: 7.4947x; 7.4947x over previous
import functools

import jax
import jax.numpy as jnp
from jax import lax
from jax.experimental import pallas as pl
from jax.experimental.pallas import tpu as pltpu
from jax.experimental.pallas import tpu_sc as plsc

_V = 100000
_D = 32
_B = 4096
_L = 50
_F = 4
_N = _B * _L

_NC = 2
_NS = 16
_NW = _NC * _NS
_PER_W = _N // _NW
_C = 128
_NCH = _PER_W // _C


def _sc_body(i0, i1, i2, i3, e0, e1, e2, e3, out, idx_v, rows_v, sem):
    wid = lax.axis_index("s") * _NC + lax.axis_index("c")
    base = wid * _PER_W

    for f, ih in enumerate((i0, i1, i2, i3)):
        pltpu.sync_copy(ih.at[wid], idx_v.at[f])

    def chunk(ci, _):
        copies = []
        for f, eh in enumerate((e0, e1, e2, e3)):
            copies.append(
                pltpu.async_copy(eh.at[idx_v.at[f, ci]], rows_v.at[f], sem))
        for c in copies:
            c.wait()
        off = base + ci * _C
        for f in range(_F):
            pltpu.sync_copy(rows_v.at[f], out.at[pl.ds(off, _C), f])
        return 0

    lax.fori_loop(0, _NCH, chunk, 0)


@functools.partial(
    pl.kernel,
    out_type=jax.ShapeDtypeStruct((_N, _F, _D), jnp.float32),
    mesh=plsc.VectorSubcoreMesh(core_axis_name="c", subcore_axis_name="s"),
    compiler_params=pltpu.CompilerParams(use_tc_tiling_on_sc=False),
    scratch_types=[
        pltpu.VMEM((_F, _NCH, _C), jnp.int32),
        pltpu.VMEM((_F, _C, _D), jnp.float32),
        pltpu.SemaphoreType.DMA,
    ],
)
def _multi_gather(i0, i1, i2, i3, e0, e1, e2, e3, out, idx_v, rows_v, sem):
    _sc_body(i0, i1, i2, i3, e0, e1, e2, e3, out, idx_v, rows_v, sem)


def kernel(idx_cat0, idx_cat1, idx_cat2, idx_cat3,
           emb_cat0, emb_cat1, emb_cat2, emb_cat3):
    idxs = [i.reshape(_NW, _NCH, _C).astype(jnp.int32)
            for i in (idx_cat0, idx_cat1, idx_cat2, idx_cat3)]
    out = _multi_gather(*idxs, emb_cat0, emb_cat1, emb_cat2, emb_cat3)
    return out.reshape(_B, _L, _F * _D)
